# Initial kernel scaffold; baseline (speedup 1.0000x reference)
import jax
import jax.numpy as jnp
from jax.experimental import pallas as pl


def kernel(item, src, value, dt, mask, lab_flag, mask_pos, item_emb, src_emb, flag_emb, vW1, vb1, vW2, vb2, dW1, db1, dW2, db2, mask_token, gamma, beta):
    raise NotImplementedError("write your pallas kernel here")


# speedup vs baseline: 3.8570x; 3.8570x over previous
import functools

import jax
import jax.numpy as jnp
from jax import lax
from jax.experimental import pallas as pl
from jax.experimental.pallas import tpu as pltpu
from jax.experimental.pallas import tpu_sc as plsc


def _make_sc_gather(vocab, d, ntok):
    info = plsc.get_sparse_core_info()
    nw = info.num_cores * info.num_subcores
    nc = info.num_cores
    tpw = ntok // nw
    chunk = 128
    nch = tpw // chunk
    mesh = plsc.VectorSubcoreMesh(core_axis_name="c", subcore_axis_name="s")

    @functools.partial(
        pl.kernel,
        mesh=mesh,
        out_type=jax.ShapeDtypeStruct((ntok, d), jnp.float32),
        scratch_types=[
            pltpu.VMEM((nch, chunk), jnp.int32),
            pltpu.VMEM((chunk, d), jnp.float32),
            pltpu.SemaphoreType.DMA,
        ],
    )
    def sc_gather(idx_hbm, table_hbm, out_hbm, idx_v, rows_v, sem):
        wid = lax.axis_index("s") * nc + lax.axis_index("c")
        pltpu.sync_copy(idx_hbm.at[wid], idx_v)

        def body(k, carry):
            pltpu.async_copy(table_hbm.at[idx_v.at[k]], rows_v, sem).wait()
            pltpu.sync_copy(rows_v, out_hbm.at[pl.ds(wid * tpw + k * chunk, chunk)])
            return carry

        lax.fori_loop(0, nch, body, 0)

    return sc_gather, nw, nch, chunk


def _tc_body(g_ref, v_ref, t_ref, src_ref, lab_ref, mp_ref, mk_ref,
             w1_ref, b1_ref, w2_ref, b2_ref, e_ref, mt_ref, gam_ref, bet_ref,
             o_ref):
    blk = g_ref.shape[0]
    f32 = jnp.float32

    def silu(z):
        return z * jax.nn.sigmoid(z)

    sv = silu(v_ref[...] * w1_ref[0:1, :] + b1_ref[0:1, :])
    st = silu(t_ref[...] * w1_ref[1:2, :] + b1_ref[1:2, :])
    s = jnp.concatenate([sv, st], axis=1)
    y = jnp.dot(s, w2_ref[...], preferred_element_type=f32)

    src = src_ref[...]
    lab = lab_ref[...]
    mapped = jnp.where(lab < 0, 1, jnp.where(lab > 0, 3, 0))
    cols = lax.broadcasted_iota(jnp.int32, (blk, 16), 1)
    oh = jnp.concatenate(
        [(src == cols).astype(f32), (mapped == cols).astype(f32)], axis=1)
    emb = jnp.dot(oh, e_ref[...], preferred_element_type=f32)

    x = g_ref[...] + y + b2_ref[...] + emb + mp_ref[...] * mt_ref[...]
    x = x * mk_ref[...]

    mu = jnp.mean(x, axis=1, keepdims=True)
    xc = x - mu
    var = jnp.mean(xc * xc, axis=1, keepdims=True)
    o_ref[...] = xc * lax.rsqrt(var + 1e-5) * gam_ref[...] + bet_ref[...]


def kernel(item, src, value, dt, mask, lab_flag, mask_pos,
           item_emb, src_emb, flag_emb,
           vW1, vb1, vW2, vb2, dW1, db1, dW2, db2,
           mask_token, gamma, beta):
    b, l = item.shape
    vocab, d = item_emb.shape
    ntok = b * l

    sc_gather, nw, nch, chunk = _make_sc_gather(vocab, d, ntok)
    idx3 = item.reshape(nw, nch, chunk)
    gathered = sc_gather(idx3, item_emb)

    blk = 2048
    nblk = ntok // blk
    f32 = jnp.float32

    col = lambda a, dt_: a.reshape(ntok, 1).astype(dt_)
    w1cat = jnp.concatenate([vW1, dW1], axis=0)
    b1cat = jnp.stack([vb1, db1], axis=0)
    w2cat = jnp.concatenate([vW2, dW2], axis=0)
    b2row = (vb2 + db2).reshape(1, d)
    ecat = jnp.zeros((32, d), f32).at[:src_emb.shape[0]].set(src_emb)
    ecat = ecat.at[16:16 + flag_emb.shape[0]].set(flag_emb)

    row_spec = pl.BlockSpec((blk, d), lambda i: (i, 0))
    col_spec = pl.BlockSpec((blk, 1), lambda i: (i, 0))
    fix = lambda shape: pl.BlockSpec(shape, lambda i: (0, 0))

    out = pl.pallas_call(
        _tc_body,
        grid=(nblk,),
        in_specs=[
            row_spec, col_spec, col_spec, col_spec, col_spec, col_spec,
            col_spec,
            fix((2, d)), fix((2, d)), fix((2 * d, d)), fix((1, d)),
            fix((32, d)), fix((1, d)), fix((1, d)), fix((1, d)),
        ],
        out_specs=row_spec,
        out_shape=jax.ShapeDtypeStruct((ntok, d), f32),
    )(gathered, col(value, f32), col(dt, f32), col(src, jnp.int32),
      col(lab_flag, jnp.int32), col(mask_pos, f32), col(mask, f32),
      w1cat, b1cat, w2cat, b2row, ecat, mask_token.reshape(1, d),
      gamma.reshape(1, d), beta.reshape(1, d))

    return out.reshape(b, l, d)


# speedup vs baseline: 6.5696x; 1.7033x over previous
import functools

import jax
import jax.numpy as jnp
from jax import lax
from jax.experimental import pallas as pl
from jax.experimental.pallas import tpu as pltpu
from jax.experimental.pallas import tpu_sc as plsc


def _make_sc_gather(d, ntok):
    info = plsc.get_sparse_core_info()
    nw = info.num_cores * info.num_subcores
    nc = info.num_cores
    tpw = ntok // nw
    chunk = 128
    nch = tpw // chunk
    mesh = plsc.VectorSubcoreMesh(core_axis_name="c", subcore_axis_name="s")

    @functools.partial(
        pl.kernel,
        mesh=mesh,
        out_type=jax.ShapeDtypeStruct((ntok, d), jnp.float32),
        scratch_types=[
            pltpu.VMEM((nch, chunk), jnp.int32),
            pltpu.VMEM((chunk, d), jnp.float32),
            pltpu.SemaphoreType.DMA,
        ],
    )
    def sc_gather(idx_hbm, table_hbm, out_hbm, idx_v, rows_v, sem):
        wid = lax.axis_index("s") * nc + lax.axis_index("c")
        pltpu.sync_copy(idx_hbm.at[wid], idx_v)

        def body(k, carry):
            pltpu.async_copy(table_hbm.at[idx_v.at[k]], rows_v, sem).wait()
            pltpu.sync_copy(rows_v, out_hbm.at[pl.ds(wid * tpw + k * chunk, chunk)])
            return carry

        lax.fori_loop(0, nch, body, 0)

    return sc_gather, nw, nch, chunk


def _tc_body(g_ref, a_ref, ci_ref,
             w1_ref, w2c_ref, ec_ref, c_ref, gb_ref, o_ref):
    blk = g_ref.shape[0]
    d = g_ref.shape[1]
    f32 = jnp.float32

    cib = jnp.dot(ci_ref[...], jnp.ones((1, d), f32),
                  preferred_element_type=f32)
    mkp = (cib >= 80.0).astype(f32)
    ohv = cib - 80.0 * mkp
    iota = lax.broadcasted_iota(jnp.int32, (blk, d), 1).astype(f32)
    oh = (ohv == iota).astype(f32)

    pre = jnp.dot(a_ref[...], w1_ref[...], preferred_element_type=f32)
    sact = pre * jax.nn.sigmoid(pre)

    xc = (jnp.dot(g_ref[...], c_ref[...], preferred_element_type=f32)
          + jnp.dot(sact, w2c_ref[...], preferred_element_type=f32)
          + jnp.dot(oh, ec_ref[...], preferred_element_type=f32))
    varp = jnp.dot(xc * xc, jnp.full((d, d), 1.0 / d, f32),
                   preferred_element_type=f32)
    sp = mkp * lax.rsqrt(mkp * varp + 1e-5)
    o_ref[...] = xc * sp * gb_ref[0:1, :] + gb_ref[1:2, :]


def kernel(item, src, value, dt, mask, lab_flag, mask_pos,
           item_emb, src_emb, flag_emb,
           vW1, vb1, vW2, vb2, dW1, db1, dW2, db2,
           mask_token, gamma, beta):
    b, l = item.shape
    d = item_emb.shape[1]
    ntok = b * l
    f32 = jnp.float32

    sc_gather, nw, nch, chunk = _make_sc_gather(d, ntok)
    idx3 = item.reshape(nw, nch, chunk)
    gathered = sc_gather(idx3, item_emb)

    blk = 2048
    nblk = ntok // blk

    a_in = jnp.stack(
        [value.reshape(-1).astype(f32), dt.reshape(-1).astype(f32),
         jnp.ones((ntok,), f32)] + [jnp.zeros((ntok,), f32)] * 5, axis=1)
    w1big = jnp.zeros((8, 2 * d), f32)
    w1big = w1big.at[0, :d].set(vW1[0]).at[1, d:].set(dW1[0])
    w1big = w1big.at[2, :d].set(vb1).at[2, d:].set(db1)
    w2cat = jnp.concatenate([vW2, dW2], axis=0)

    combo = (src_emb[:, None, None, :] + flag_emb[None, :, None, :]
             + jnp.arange(2, dtype=f32)[None, None, :, None] * mask_token
             + (vb2 + db2)).reshape(-1, d)
    ecat = jnp.zeros((128, d), f32).at[:combo.shape[0]].set(combo)
    mapped = jnp.where(lab_flag < 0, 1, jnp.where(lab_flag > 0, 3, 0))
    cidx = (src * 8 + mapped * 2 + mask_pos
            + 80 * mask.astype(jnp.int32)).reshape(ntok, 1).astype(f32)

    cmat = jnp.eye(d, dtype=f32) - 1.0 / d
    w2c = jnp.dot(w2cat, cmat)
    ec = jnp.dot(ecat, cmat)
    gb = jnp.stack([gamma, beta], axis=0)

    row_spec = pl.BlockSpec((blk, d), lambda i: (i, 0))
    fix = lambda shape: pl.BlockSpec(shape, lambda i: (0, 0))

    out = pl.pallas_call(
        _tc_body,
        grid=(nblk,),
        in_specs=[
            row_spec, pl.BlockSpec((blk, 8), lambda i: (i, 0)),
            pl.BlockSpec((blk, 1), lambda i: (i, 0)),
            fix((8, 2 * d)), fix((2 * d, d)), fix((128, d)), fix((d, d)),
            fix((2, d)),
        ],
        out_specs=row_spec,
        out_shape=jax.ShapeDtypeStruct((ntok, d), f32),
    )(gathered, a_in, cidx, w1big, w2c, ec, cmat, gb)

    return out.reshape(b, l, d)


# speedup vs baseline: 8.6295x; 1.3136x over previous
import functools

import jax
import jax.numpy as jnp
from jax import lax
from jax.experimental import pallas as pl
from jax.experimental.pallas import tpu as pltpu
from jax.experimental.pallas import tpu_sc as plsc


def _make_sc_gather(d, ntok):
    info = plsc.get_sparse_core_info()
    nw = info.num_cores * info.num_subcores
    nc = info.num_cores
    tpw = ntok // nw
    chunk = 128
    nch = tpw // chunk
    mesh = plsc.VectorSubcoreMesh(core_axis_name="c", subcore_axis_name="s")

    @functools.partial(
        pl.kernel,
        mesh=mesh,
        out_type=jax.ShapeDtypeStruct((ntok, d), jnp.float32),
        scratch_types=[
            pltpu.VMEM((nch, chunk), jnp.int32),
            pltpu.VMEM((chunk, d), jnp.float32),
            pltpu.SemaphoreType.DMA,
        ],
    )
    def sc_gather(idx_hbm, table_hbm, out_hbm, idx_v, rows_v, sem):
        wid = lax.axis_index("s") * nc + lax.axis_index("c")
        pltpu.sync_copy(idx_hbm.at[wid], idx_v)

        def body(k, carry):
            pltpu.async_copy(table_hbm.at[idx_v.at[k]], rows_v, sem).wait()
            pltpu.sync_copy(rows_v, out_hbm.at[pl.ds(wid * tpw + k * chunk, chunk)])
            return carry

        lax.fori_loop(0, nch, body, 0)

    return sc_gather, nw, nch, chunk


def _tc_body(g_ref, s_ref, w1_ref, w2c_ref, ec_ref, c_ref, gb_ref, o_ref):
    blk = g_ref.shape[0]
    d = g_ref.shape[1]
    f32 = jnp.float32

    p = lax.dot_general(s_ref[...], w1_ref[...], (((0,), (0,)), ((), ())),
                        preferred_element_type=f32)
    pre = p[:, :2 * d]
    cib = p[:, 2 * d:]
    mkp = (cib >= 80.0).astype(f32)
    ohv = cib - 80.0 * mkp
    iota = lax.broadcasted_iota(jnp.int32, (blk, d), 1).astype(f32)
    oh = (ohv == iota).astype(f32)

    sact = pre * jax.nn.sigmoid(pre)

    xc = (jnp.dot(g_ref[...], c_ref[...], preferred_element_type=f32)
          + jnp.dot(sact, w2c_ref[...], preferred_element_type=f32)
          + jnp.dot(oh, ec_ref[...], preferred_element_type=f32))
    varp = jnp.dot(xc * xc, jnp.full((d, d), 1.0 / d, f32),
                   preferred_element_type=f32)
    sp = mkp * lax.rsqrt(mkp * varp + 1e-5)
    o_ref[...] = xc * sp * gb_ref[0:1, :] + gb_ref[1:2, :]


def kernel(item, src, value, dt, mask, lab_flag, mask_pos,
           item_emb, src_emb, flag_emb,
           vW1, vb1, vW2, vb2, dW1, db1, dW2, db2,
           mask_token, gamma, beta):
    b, l = item.shape
    d = item_emb.shape[1]
    ntok = b * l
    f32 = jnp.float32

    sc_gather, nw, nch, chunk = _make_sc_gather(d, ntok)
    idx3 = item.reshape(nw, nch, chunk)
    gathered = sc_gather(idx3, item_emb)

    blk = 2048
    nblk = ntok // blk

    mapped = jnp.where(lab_flag < 0, 1, jnp.where(lab_flag > 0, 3, 0))
    cidx = (src * 8 + mapped * 2 + mask_pos
            + 80 * mask.astype(jnp.int32)).astype(f32)
    scal = jnp.stack(
        [value.reshape(-1).astype(f32), dt.reshape(-1).astype(f32),
         jnp.ones((ntok,), f32), cidx.reshape(-1)], axis=0)

    w1big = jnp.zeros((4, 3 * d), f32)
    w1big = w1big.at[0, :d].set(vW1[0]).at[1, d:2 * d].set(dW1[0])
    w1big = w1big.at[2, :d].set(vb1).at[2, d:2 * d].set(db1)
    w1big = w1big.at[3, 2 * d:].set(1.0)
    w2cat = jnp.concatenate([vW2, dW2], axis=0)

    combo = (src_emb[:, None, None, :] + flag_emb[None, :, None, :]
             + jnp.arange(2, dtype=f32)[None, None, :, None] * mask_token
             + (vb2 + db2)).reshape(-1, d)
    ecat = jnp.zeros((128, d), f32).at[:combo.shape[0]].set(combo)

    cmat = jnp.eye(d, dtype=f32) - 1.0 / d
    w2c = jnp.dot(w2cat, cmat)
    ec = jnp.dot(ecat, cmat)
    gb = jnp.stack([gamma, beta], axis=0)

    row_spec = pl.BlockSpec((blk, d), lambda i: (i, 0))
    fix = lambda shape: pl.BlockSpec(shape, lambda i: (0, 0))

    out = pl.pallas_call(
        _tc_body,
        grid=(nblk,),
        in_specs=[
            row_spec, pl.BlockSpec((4, blk), lambda i: (0, i)),
            fix((4, 3 * d)), fix((2 * d, d)), fix((128, d)), fix((d, d)),
            fix((2, d)),
        ],
        out_specs=row_spec,
        out_shape=jax.ShapeDtypeStruct((ntok, d), f32),
    )(gathered, scal, w1big, w2c, ec, cmat, gb)

    return out.reshape(b, l, d)


# speedup vs baseline: 10.7733x; 1.2484x over previous
import functools

import jax
import jax.numpy as jnp
from jax import lax
from jax.experimental import pallas as pl
from jax.experimental.pallas import tpu as pltpu
from jax.experimental.pallas import tpu_sc as plsc


def _make_sc_gather(d, ntok):
    info = plsc.get_sparse_core_info()
    nw = info.num_cores * info.num_subcores
    nc = info.num_cores
    tpw = ntok // nw
    chunk = 128
    nch = tpw // chunk
    mesh = plsc.VectorSubcoreMesh(core_axis_name="c", subcore_axis_name="s")

    @functools.partial(
        pl.kernel,
        mesh=mesh,
        out_type=jax.ShapeDtypeStruct((ntok, d), jnp.float32),
        scratch_types=[
            pltpu.VMEM((nch, chunk), jnp.int32),
            pltpu.VMEM((chunk, d), jnp.float32),
            pltpu.SemaphoreType.DMA,
        ],
    )
    def sc_gather(idx_hbm, table_hbm, out_hbm, idx_v, rows_v, sem):
        wid = lax.axis_index("s") * nc + lax.axis_index("c")
        pltpu.sync_copy(idx_hbm.at[wid], idx_v)

        def body(k, carry):
            pltpu.async_copy(table_hbm.at[idx_v.at[k]], rows_v, sem).wait()
            pltpu.sync_copy(rows_v, out_hbm.at[pl.ds(wid * tpw + k * chunk, chunk)])
            return carry

        lax.fori_loop(0, nch, body, 0)

    return sc_gather, nw, nch, chunk


def _tc_body(g_ref, s_ref, w1_ref, w2c_ref, ec_ref, c_ref, gb_ref, o_ref):
    blk = g_ref.shape[0]
    d = g_ref.shape[1]
    f32 = jnp.float32

    p = lax.dot_general(s_ref[...], w1_ref[...], (((0,), (0,)), ((), ())),
                        preferred_element_type=f32)
    pre = p[:, :2 * d]
    cib = p[:, 2 * d:]
    mkp = (cib >= 80.0).astype(f32)
    ohv = cib - 80.0 * mkp
    iota = lax.broadcasted_iota(jnp.int32, (blk, d), 1).astype(f32)
    oh = (ohv == iota).astype(f32)

    sact = pre * jax.nn.sigmoid(pre)

    xc = (jnp.dot(g_ref[...], c_ref[...], preferred_element_type=f32)
          + jnp.dot(sact, w2c_ref[...], preferred_element_type=f32)
          + jnp.dot(oh, ec_ref[...], preferred_element_type=f32))
    varp = jnp.dot(xc * xc, jnp.full((d, d), 1.0 / d, f32),
                   preferred_element_type=f32)
    sp = mkp * lax.rsqrt(mkp * varp + 1e-5)
    o_ref[...] = xc * sp * gb_ref[0:1, :] + gb_ref[1:2, :]


def _tc_body_alias(g_ref, s_ref, w1_ref, w2c_ref, ec_ref, c_ref, gb_ref,
                   prev_ref, o_ref):
    del prev_ref
    _tc_body(g_ref, s_ref, w1_ref, w2c_ref, ec_ref, c_ref, gb_ref, o_ref)


def kernel(item, src, value, dt, mask, lab_flag, mask_pos,
           item_emb, src_emb, flag_emb,
           vW1, vb1, vW2, vb2, dW1, db1, dW2, db2,
           mask_token, gamma, beta):
    b, l = item.shape
    d = item_emb.shape[1]
    ntok = b * l
    f32 = jnp.float32

    nslice = 5
    tok_s = ntok // nslice
    sc_gather, nw, nch, chunk = _make_sc_gather(d, tok_s)
    idx4 = item.reshape(nslice, nw, nch, chunk)
    gs = [sc_gather(idx4[s], item_emb) for s in range(nslice)]

    blk = 2048
    nblk_s = tok_s // blk

    mapped = jnp.where(lab_flag < 0, 1, jnp.where(lab_flag > 0, 3, 0))
    cidx = (src * 8 + mapped * 2 + mask_pos
            + 80 * mask.astype(jnp.int32)).astype(f32)
    scal = jnp.stack(
        [value.reshape(-1).astype(f32), dt.reshape(-1).astype(f32),
         jnp.ones((ntok,), f32), cidx.reshape(-1)], axis=0)

    w1big = jnp.zeros((4, 3 * d), f32)
    w1big = w1big.at[0, :d].set(vW1[0]).at[1, d:2 * d].set(dW1[0])
    w1big = w1big.at[2, :d].set(vb1).at[2, d:2 * d].set(db1)
    w1big = w1big.at[3, 2 * d:].set(1.0)
    w2cat = jnp.concatenate([vW2, dW2], axis=0)

    combo = (src_emb[:, None, None, :] + flag_emb[None, :, None, :]
             + jnp.arange(2, dtype=f32)[None, None, :, None] * mask_token
             + (vb2 + db2)).reshape(-1, d)
    ecat = jnp.zeros((128, d), f32).at[:combo.shape[0]].set(combo)

    cmat = jnp.eye(d, dtype=f32) - 1.0 / d
    w2c = jnp.dot(w2cat, cmat)
    ec = jnp.dot(ecat, cmat)
    gb = jnp.stack([gamma, beta], axis=0)

    fix = lambda shape: pl.BlockSpec(shape, lambda i: (0, 0))
    fixed_specs = [fix((4, 3 * d)), fix((2 * d, d)), fix((128, d)),
                   fix((d, d)), fix((2, d))]

    out = None
    for s in range(nslice):
        off = s * nblk_s
        in_specs = [
            pl.BlockSpec((blk, d), lambda i: (i, 0)),
            pl.BlockSpec((4, blk), lambda i, off=off: (0, i + off)),
        ] + list(fixed_specs)
        args = [gs[s], scal, w1big, w2c, ec, cmat, gb]
        if s == 0:
            body, aliases = _tc_body, {}
        else:
            in_specs.append(fix((8, d)))
            args.append(out)
            body, aliases = _tc_body_alias, {7: 0}
        out = pl.pallas_call(
            body,
            grid=(nblk_s,),
            in_specs=in_specs,
            out_specs=pl.BlockSpec((blk, d), lambda i, off=off: (i + off, 0)),
            out_shape=jax.ShapeDtypeStruct((ntok, d), f32),
            input_output_aliases=aliases,
        )(*args)

    return out.reshape(b, l, d)
